# baseline (speedup 1.0000x reference)
import functools

import jax
import jax.numpy as jnp
from jax import lax
from jax.experimental import pallas as pl
from jax.experimental.pallas import tpu as pltpu
from jax.experimental.pallas import tpu_sc as plsc

B = 4096
G = 12001
K = 8
NW = 32
RPW = B // NW
BANDS = RPW // 8
NBLK = 94

NEG = -3.0e38
POS = 3.0e38
SENT = -1e9
BIGIDX = 1 << 30

def _gather(x, idx):
    return lax.gather(
        x, idx.reshape(16, 1),
        dimension_numbers=lax.GatherDimensionNumbers(
            offset_dims=(), collapsed_slice_dims=(0,), start_index_map=(0,)),
        slice_sizes=(1,),
        mode=lax.GatherScatterMode.PROMISE_IN_BOUNDS)


def _any(mask):
    return jnp.max(mask.astype(jnp.int32)) > 0


def _sc_body(spec, theta_out, succ_out, buf, tbuf, sbuf, sem):
    wid = lax.axis_index("s") * 2 + lax.axis_index("c")
    lane = lax.iota(jnp.int32, 16)
    low8 = lane < 8
    def chunk_fn(k, t, i, tv, ti, thresh):
        col = 128 * t + 16 * k
        xk = buf[i, pl.ds(col, 16)]
        if k == 0:
            blv = buf[i, pl.ds(jnp.maximum(128 * t - 16, 0), 16)]
            bl = jnp.where(t > 0, blv[15], POS)
            xl = jnp.where(lane == 0, bl, _gather(xk, jnp.maximum(lane - 1, 0)))
        else:
            xl = buf[i, pl.ds(col - 1, 16)]
        if k == 7:
            brv = buf[i, pl.ds(jnp.minimum(128 * (t + 1), 12016), 16)]
            br = jnp.where(t < NBLK - 1, brv[0], NEG)
            xr = jnp.where(lane == 15, br,
                           _gather(xk, jnp.minimum(lane + 1, 15)))
        else:
            xr = buf[i, pl.ds(col + 1, 16)]
        gidx = col + lane
        ip = (xk >= xl) & (xr <= xk) & (gidx >= 1) & (gidx <= G - 2)
        pv = jnp.where(ip, xk, SENT)
        cand = (gidx <= G - 1) & (pv > thresh)

        def ins_cond(c):
            return _any(c[0])

        def ins_body(c):
            cnd, tv_, ti_, th_ = c
            f = plsc.all_reduce_ffs(cnd)
            sel = lane == f
            cv = jnp.max(jnp.where(sel, pv, NEG))
            ci = jnp.max(jnp.where(sel, gidx, -1))
            ok = cv > th_
            vmin = jnp.min(tv_)
            evt = jnp.max(jnp.where(tv_ == vmin, ti_, -BIGIDX))
            esel = (ti_ == evt) & ok
            tv2 = jnp.where(esel, cv, tv_)
            ti2 = jnp.where(esel, ci, ti_)
            th2 = jnp.where(ok, jnp.min(tv2), th_)
            cnd2 = cnd & (lane != f) & (pv > th2)
            return (cnd2, tv2, ti2, th2)

        _, tv, ti, thresh = lax.while_loop(
            ins_cond, ins_body, (cand, tv, ti, thresh))
        return tv, ti, thresh

    def process_row(i, bl_idx):
        def block_body(t, carry):
            tv, ti, thresh = carry
            col0 = 128 * t
            xs = [buf[i, pl.ds(col0 + 16 * k, 16)] for k in range(8)]
            m = jnp.maximum(
                jnp.maximum(jnp.maximum(xs[0], xs[1]),
                            jnp.maximum(xs[2], xs[3])),
                jnp.maximum(jnp.maximum(xs[4], xs[5]),
                            jnp.maximum(xs[6], xs[7])))
            hit = _any(m > thresh)

            def trig(tv, ti, thresh):
                for k in range(8):
                    xk = buf[i, pl.ds(col0 + 16 * k, 16)]
                    sub_hit = _any(xk > thresh)
                    tv, ti, thresh = lax.cond(
                        sub_hit,
                        lambda tv_, ti_, th_, kk=k: chunk_fn(
                            kk, t, i, tv_, ti_, th_),
                        lambda tv_, ti_, th_: (tv_, ti_, th_),
                        tv, ti, thresh)
                return tv, ti, thresh

            return lax.cond(
                hit, trig, lambda tv_, ti_, th_: (tv_, ti_, th_),
                tv, ti, thresh)

        tv0 = jnp.where(low8, NEG, POS)
        ti0 = jnp.where(low8, lane - 8, BIGIDX)
        tv, ti, _ = lax.fori_loop(0, NBLK, block_body,
                                  (tv0, ti0, jnp.float32(NEG)))

        ti_s, tv_s = plsc.sort_key_val(ti, tv)
        theta = -60.0 + 0.01 * ti_s.astype(jnp.float32)
        last_val = jnp.max(jnp.where(lane == 7, tv_s, NEG))
        sflag = jnp.where(last_val > 0.0, jnp.float32(1.0), jnp.float32(0.0))
        lrow = bl_idx * 8 + i
        tbuf[pl.ds(lrow * K, 16)] = theta
        sbuf[pl.ds(lrow, 16)] = jnp.where(lane == 0, sflag, jnp.float32(0.0))

    def band_loop(bl_idx, c):
        r8 = (wid * BANDS + bl_idx) * 8
        cp = pltpu.make_async_copy(spec.at[pl.ds(r8, 8)], buf, sem)
        cp.start()
        cp.wait()

        def row_loop(i, c2):
            process_row(i, bl_idx)
            return c2

        return lax.fori_loop(0, 8, row_loop, c)

    lax.fori_loop(0, BANDS, band_loop, 0)

    pltpu.sync_copy(tbuf.at[pl.ds(0, RPW * K)],
                    theta_out.at[pl.ds(wid * RPW * K, RPW * K)])
    pltpu.sync_copy(sbuf.at[pl.ds(0, RPW)],
                    succ_out.at[pl.ds(wid * RPW, RPW)])


@jax.jit
def _run(spectrum):
    mesh = plsc.VectorSubcoreMesh(core_axis_name="c", subcore_axis_name="s")
    f = functools.partial(
        pl.kernel,
        mesh=mesh,
        compiler_params=pltpu.CompilerParams(needs_layout_passes=False),
        out_type=[
            jax.ShapeDtypeStruct((B * K,), jnp.float32),
            jax.ShapeDtypeStruct((B,), jnp.float32),
        ],
        scratch_types=[
            pltpu.VMEM((8, G), jnp.float32),
            pltpu.VMEM((RPW * K + 16,), jnp.float32),
            pltpu.VMEM((RPW + 16,), jnp.float32),
            pltpu.SemaphoreType.DMA,
        ],
    )(_sc_body)
    theta_flat, succ = f(spectrum)
    return theta_flat.reshape(B, K), succ


def kernel(spectrum, k, min_sep):
    theta, succ = _run(spectrum)
    return succ != 0.0, theta


# speedup vs baseline: 1.2088x; 1.2088x over previous
import functools

import jax
import jax.numpy as jnp
from jax import lax
from jax.experimental import pallas as pl
from jax.experimental.pallas import tpu as pltpu
from jax.experimental.pallas import tpu_sc as plsc

B = 4096
G = 12001
K = 8
NW = 32
RPW = B // NW
BANDS = RPW // 8
NBLK = 94

NEG = -3.0e38
POS = 3.0e38
SENT = -1e9
BIGIDX = 1 << 30


def _gather(x, idx):
    return lax.gather(
        x, idx.reshape(16, 1),
        dimension_numbers=lax.GatherDimensionNumbers(
            offset_dims=(), collapsed_slice_dims=(0,), start_index_map=(0,)),
        slice_sizes=(1,),
        mode=lax.GatherScatterMode.PROMISE_IN_BOUNDS)


def _sc_body(spec, theta_out, succ_out, buf, tbuf, sbuf, sem):
    wid = lax.axis_index("s") * 2 + lax.axis_index("c")
    lane = lax.iota(jnp.int32, 16)
    low8 = lane < 8

    def splat_min(v):
        for s in (1, 2, 4, 8):
            v = jnp.minimum(v, _gather(v, lane ^ s))
        return v

    def splat_max(v):
        for s in (1, 2, 4, 8):
            v = jnp.maximum(v, _gather(v, lane ^ s))
        return v

    def any_lanes(mask):
        return plsc.all_reduce_population_count(mask)[0] > 0

    def chunk_fn(k, t, i, tv, ti, thv):
        col = 128 * t + 16 * k
        xk = buf[i, pl.ds(col, 16)]
        if k == 0:
            blv = buf[i, pl.ds(jnp.maximum(128 * t - 16, 0), 16)]
            bl = jnp.where(t > 0, blv[15], POS)
            xl = jnp.where(lane == 0, bl, _gather(xk, jnp.maximum(lane - 1, 0)))
        else:
            xl = buf[i, pl.ds(col - 1, 16)]
        if k == 7:
            brv = buf[i, pl.ds(jnp.minimum(128 * (t + 1), 12016), 16)]
            br = jnp.where(t < NBLK - 1, brv[0], NEG)
            xr = jnp.where(lane == 15, br,
                           _gather(xk, jnp.minimum(lane + 1, 15)))
        else:
            xr = buf[i, pl.ds(col + 1, 16)]
        gidx = col + lane
        ip = (xk >= xl) & (xr <= xk) & (gidx >= 1) & (gidx <= G - 2)
        pv = jnp.where(ip, xk, SENT)
        cand = (gidx <= G - 1) & (pv > thv)

        def ins_cond(c):
            return any_lanes(c[0])

        def ins_body(c):
            cnd, tv_, ti_, th_ = c
            f = plsc.all_reduce_ffs(cnd)
            cv = _gather(pv, f)
            ci = _gather(gidx, f)
            okv = cv > th_
            vmin = splat_min(tv_)
            evt = splat_max(jnp.where(tv_ == vmin, ti_, -BIGIDX))
            esel = (ti_ == evt) & okv
            tv2 = jnp.where(esel, cv, tv_)
            ti2 = jnp.where(esel, ci, ti_)
            th2 = jnp.where(okv, splat_min(tv2), th_)
            cnd2 = cnd & (lane != f) & (pv > th2)
            return (cnd2, tv2, ti2, th2)

        _, tv, ti, thv = lax.while_loop(
            ins_cond, ins_body, (cand, tv, ti, thv))
        return tv, ti, thv

    def process_row(i, bl_idx):
        def block_body(t, carry):
            tv, ti, thv = carry
            col0 = 128 * t
            xs = [buf[i, pl.ds(col0 + 16 * k, 16)] for k in range(8)]
            m = jnp.maximum(
                jnp.maximum(jnp.maximum(xs[0], xs[1]),
                            jnp.maximum(xs[2], xs[3])),
                jnp.maximum(jnp.maximum(xs[4], xs[5]),
                            jnp.maximum(xs[6], xs[7])))
            hit = any_lanes(m > thv)

            def trig(tv, ti, thv):
                for k in range(8):
                    xk = buf[i, pl.ds(col0 + 16 * k, 16)]
                    sub_hit = any_lanes(xk > thv)
                    tv, ti, thv = lax.cond(
                        sub_hit,
                        lambda tv_, ti_, th_, kk=k: chunk_fn(
                            kk, t, i, tv_, ti_, th_),
                        lambda tv_, ti_, th_: (tv_, ti_, th_),
                        tv, ti, thv)
                return tv, ti, thv

            return lax.cond(
                hit, trig, lambda tv_, ti_, th_: (tv_, ti_, th_),
                tv, ti, thv)

        tv0 = jnp.where(low8, NEG, POS)
        ti0 = jnp.where(low8, lane - 8, BIGIDX)
        th0 = jnp.full((16,), NEG, jnp.float32)
        tv, ti, _ = lax.fori_loop(0, NBLK, block_body, (tv0, ti0, th0))

        ti_s, tv_s = plsc.sort_key_val(ti, tv)
        theta = -60.0 + 0.01 * ti_s.astype(jnp.float32)
        sflag = jnp.where(tv_s[7] > 0.0, jnp.float32(1.0), jnp.float32(0.0))
        lrow = bl_idx * 8 + i
        tbuf[pl.ds(lrow * K, 16)] = theta
        sbuf[pl.ds(lrow, 16)] = jnp.where(lane == 0, sflag, jnp.float32(0.0))

    def band_loop(bl_idx, c):
        r8 = (wid * BANDS + bl_idx) * 8
        cp = pltpu.make_async_copy(spec.at[pl.ds(r8, 8)], buf, sem)
        cp.start()
        cp.wait()

        def row_loop(i, c2):
            process_row(i, bl_idx)
            return c2

        return lax.fori_loop(0, 8, row_loop, c)

    lax.fori_loop(0, BANDS, band_loop, 0)

    pltpu.sync_copy(tbuf.at[pl.ds(0, RPW * K)],
                    theta_out.at[pl.ds(wid * RPW * K, RPW * K)])
    pltpu.sync_copy(sbuf.at[pl.ds(0, RPW)],
                    succ_out.at[pl.ds(wid * RPW, RPW)])


@jax.jit
def _run(spectrum):
    mesh = plsc.VectorSubcoreMesh(core_axis_name="c", subcore_axis_name="s")
    f = functools.partial(
        pl.kernel,
        mesh=mesh,
        compiler_params=pltpu.CompilerParams(needs_layout_passes=False),
        out_type=[
            jax.ShapeDtypeStruct((B * K,), jnp.float32),
            jax.ShapeDtypeStruct((B,), jnp.float32),
        ],
        scratch_types=[
            pltpu.VMEM((8, G), jnp.float32),
            pltpu.VMEM((RPW * K + 16,), jnp.float32),
            pltpu.VMEM((RPW + 16,), jnp.float32),
            pltpu.SemaphoreType.DMA,
        ],
    )(_sc_body)
    theta_flat, succ = f(spectrum)
    return theta_flat.reshape(B, K), succ


def kernel(spectrum, k, min_sep):
    theta, succ = _run(spectrum)
    return succ != 0.0, theta
